# Initial kernel scaffold; baseline (speedup 1.0000x reference)
import jax
import jax.numpy as jnp
from jax.experimental import pallas as pl


def kernel(nodes, targets, layer_predict, neighbors, feat_tables, W1, W2, Wa, Wlr, blr):
    raise NotImplementedError("write your pallas kernel here")


# speedup vs baseline: 5.7383x; 5.7383x over previous
import jax
import jax.numpy as jnp
import numpy as np
from jax import lax
from jax.experimental import pallas as pl
from jax.experimental.pallas import tpu as pltpu
from jax.experimental.pallas import tpu_sc as plsc

LNUM = 2
N = 50000
F = 128
H = 128
D = 128
B = 4096
S = 10

NC, NS = 2, 16
NW = NC * NS
BPW = B // NW
W1S = BPW * S
INV = 1.0 / (S + 1.0)
_CS = 16


_GBLK = 1000


def _g_body(feat_ref, w1_ref, g0_ref, g1_ref):
    dn = (((1,), (1,)), ((), ()))
    g0_ref[...] = lax.dot_general(
        feat_ref[0], w1_ref[0], dn, precision=lax.Precision.HIGHEST,
        preferred_element_type=jnp.float32) * INV
    g1_ref[...] = lax.dot_general(
        feat_ref[1], w1_ref[1], dn, precision=lax.Precision.HIGHEST,
        preferred_element_type=jnp.float32) * INV


def _g_precompute(feat_tables, W1):
    return pl.pallas_call(
        _g_body,
        grid=(N // _GBLK,),
        in_specs=[
            pl.BlockSpec((LNUM, _GBLK, F), lambda i: (0, i, 0)),
            pl.BlockSpec((LNUM, H, F), lambda i: (0, 0, 0)),
        ],
        out_specs=[
            pl.BlockSpec((_GBLK, H), lambda i: (i, 0)),
            pl.BlockSpec((_GBLK, H), lambda i: (i, 0)),
        ],
        out_shape=[
            jax.ShapeDtypeStruct((N, H), jnp.float32),
            jax.ShapeDtypeStruct((N, H), jnp.float32),
        ],
    )(feat_tables, W1)


def _sum_chunk_item(selfg, nbg, i):
    out = []
    for v in range(H // 16):
        t = selfg[i, pl.ds(v * 16, 16)]
        for q in range(S):
            t = t + nbg[q * _CS + i, pl.ds(v * 16, 16)]
        out.append(jnp.maximum(t, 0.0))
    return out


def _zero_buf(buf):
    z = jnp.zeros((16,), jnp.float32)

    def zrow(i, c):
        for v in range(H // 16):
            buf[i, pl.ds(v * 16, 16)] = z
        return c

    lax.fori_loop(0, BPW, zrow, 0)


def _g_chunk_dmas(gt, self_idx, nb_idx_src, nb_base, selfg, nbg, sem):
    ds = [pltpu.make_async_copy(gt.at[self_idx], selfg, sem)]
    for s in range(S):
        off = pl.multiple_of(nb_base(s), _CS)
        ds.append(pltpu.make_async_copy(
            gt.at[nb_idx_src.at[pl.ds(off, _CS)]],
            nbg.at[pl.ds(s * _CS, _CS)], sem))
    return ds


def _extract_ids(src_ids, idrows, rowidx, dst, dst_base, dst_stride):
    iota = lax.broadcasted_iota(jnp.int32, (16,), 0)

    def chunk(t, c):
        toff = pl.multiple_of(t * 16, 16)
        v = src_ids[pl.ds(toff, 16)]
        colb = (v & 7) * 16
        rows = iota + toff
        for s in range(S):
            ids = plsc.load_gather(idrows, [rows, colb + s])
            off = pl.multiple_of(dst_base + s * dst_stride + toff, 16)
            dst[pl.ds(off, 16)] = ids
        return c

    lax.fori_loop(0, BPW // 16, chunk, 0)


def _fill_rowidx(src_ids, src_off, rowidx):
    def chunk(t, c):
        toff = pl.multiple_of(t * 16, 16)
        v = src_ids[pl.ds(src_off + toff, 16)]
        rowidx[pl.ds(toff, 16)] = lax.shift_right_logical(v, 3)
        return c

    lax.fori_loop(0, BPW // 16, chunk, 0)


def _sc_body(nodes_hbm, pk0, pk1, g0, g1, hs0, hs1, h20, h21,
             nodes_v, nb1w, nb2w, idrows, rowidx, selfg, nbg, hbuf, sem):
    wid = lax.axis_index("s") * NC + lax.axis_index("c")
    base = pl.multiple_of(wid * BPW, BPW)
    pltpu.sync_copy(nodes_hbm.at[pl.ds(base, BPW)], nodes_v)

    for pk, gt, hs_out, h2_out in ((pk0, g0, hs0, h20),
                                   (pk1, g1, hs1, h21)):
        _fill_rowidx(nodes_v, 0, rowidx)
        pltpu.sync_copy(pk.at[rowidx], idrows)
        _extract_ids(nodes_v, idrows, rowidx, nb1w, 0, BPW)

        def nb2_step(k, c, pk=pk):
            koff = pl.multiple_of(k * BPW, BPW)
            _fill_rowidx(nb1w, koff, rowidx)
            pltpu.sync_copy(pk.at[rowidx], idrows)
            _extract_ids(nb1w.at[pl.ds(koff, BPW)], idrows, rowidx,
                         nb2w, koff, W1S)
            return c

        lax.fori_loop(0, W1S // BPW, nb2_step, 0)

        def self_chunk(c, carry, gt=gt):
            off = pl.multiple_of(c * _CS, _CS)
            dmas = _g_chunk_dmas(
                gt, nodes_v.at[pl.ds(off, _CS)], nb1w,
                lambda s, off=off: s * BPW + off, selfg, nbg, sem)
            for d in dmas:
                d.start()
            for d in dmas:
                d.wait()

            def item(i, cc, off=off):
                vals = _sum_chunk_item(selfg, nbg, i)
                row = off + i
                for v in range(H // 16):
                    hbuf[row, pl.ds(v * 16, 16)] = vals[v]
                return cc

            lax.fori_loop(0, _CS, item, 0)
            return carry

        lax.fori_loop(0, BPW // _CS, self_chunk, 0)
        pltpu.sync_copy(hbuf, hs_out.at[pl.ds(base, BPW)])

        _zero_buf(hbuf)

        def nb_chunk(u, carry, gt=gt):
            j0 = pl.multiple_of(u * _CS, _CS)
            dmas = _g_chunk_dmas(
                gt, nb1w.at[pl.ds(j0, _CS)], nb2w,
                lambda s, j0=j0: s * W1S + j0, selfg, nbg, sem)
            for d in dmas:
                d.start()
            for d in dmas:
                d.wait()

            def item(i, cc, u=u):
                vals = _sum_chunk_item(selfg, nbg, i)
                brow = lax.rem(u, BPW // _CS) * _CS + i
                for v in range(H // 16):
                    sl = pl.ds(v * 16, 16)
                    hbuf[brow, sl] = hbuf[brow, sl] + vals[v]
                return cc

            lax.fori_loop(0, _CS, item, 0)
            return carry

        lax.fori_loop(0, W1S // _CS, nb_chunk, 0)
        pltpu.sync_copy(hbuf, h2_out.at[pl.ds(base, BPW)])


def _sc_aggregate(nodes, pk0, pk1, g0, g1):
    mesh = plsc.VectorSubcoreMesh(
        core_axis_name="c", subcore_axis_name="s",
        num_cores=NC, num_subcores=NS)
    out = jax.ShapeDtypeStruct((B, H), jnp.float32)
    f = pl.kernel(
        _sc_body,
        out_type=(out, out, out, out),
        mesh=mesh,
        scratch_types=[
            pltpu.VMEM((BPW,), jnp.int32),
            pltpu.VMEM((W1S,), jnp.int32),
            pltpu.VMEM((W1S * S,), jnp.int32),
            pltpu.VMEM((BPW, 128), jnp.int32),
            pltpu.VMEM((BPW,), jnp.int32),
            pltpu.VMEM((_CS, H), jnp.float32),
            pltpu.VMEM((S * _CS, H), jnp.float32),
            pltpu.VMEM((BPW, H), jnp.float32),
            pltpu.SemaphoreType.DMA,
        ],
        compiler_params=pltpu.CompilerParams(needs_layout_passes=False),
    )
    return f(nodes, pk0, pk1, g0, g1)


def _fin_body(hs0, hs1, h20, h21, w2_ref, wa_ref, wlr_ref, blr_ref,
              tgt_ref, lp_ref, loss_ref, pred_ref):
    dn_t = (((1,), (1,)), ((), ()))
    dn = (((1,), (0,)), ((), ()))
    hp = lax.Precision.HIGHEST
    lp = lp_ref[0]

    embs = []
    for l, (hs, h2) in enumerate(((hs0, h20), (hs1, h21))):
        agg2 = (h2[...] + hs[...]) * INV
        emb = jnp.maximum(
            lax.dot_general(agg2, w2_ref[l], dn_t, precision=hp,
                            preferred_element_type=jnp.float32), 0.0)
        embs.append(emb)

    q = jnp.where(lp == 0, embs[0], embs[1])
    tq = jnp.tanh(q)
    scale = 1.0 / np.sqrt(D)
    scores = []
    for l in range(LNUM):
        proj = lax.dot_general(embs[l], wa_ref[...], dn, precision=hp,
                               preferred_element_type=jnp.float32)
        scores.append(
            jnp.sum(tq * jnp.tanh(proj), axis=1, keepdims=True) * scale)

    m = jnp.maximum(scores[0], scores[1])
    e0 = jnp.exp(scores[0] - m)
    e1 = jnp.exp(scores[1] - m)
    denom = e0 + e1
    fused = (e0 * embs[0] + e1 * embs[1]) / denom

    wlr = jnp.where(lp == 0, wlr_ref[0], wlr_ref[1])
    blr = jnp.where(lp == 0, blr_ref[0, 0], blr_ref[1, 0])
    logits = jnp.sum(fused * wlr, axis=1, keepdims=True) + blr
    pred = 1.0 / (1.0 + jnp.exp(-logits))
    pred_ref[...] = pred

    eps = 1e-7
    p = jnp.clip(pred, eps, 1.0 - eps)
    t = tgt_ref[...]
    ll = t * jnp.log(p) + (1.0 - t) * jnp.log(1.0 - p)
    loss_ref[...] = (-jnp.sum(ll) * (1.0 / B)).reshape(1, 1)


def _finalize(hs0, hs1, h20, h21, W2, Wa, Wlr, blr, targets, lp):
    return pl.pallas_call(
        _fin_body,
        in_specs=[pl.BlockSpec((B, H), lambda: (0, 0))] * 4 + [
            pl.BlockSpec((LNUM, D, H), lambda: (0, 0, 0)),
            pl.BlockSpec((D, D), lambda: (0, 0)),
            pl.BlockSpec((LNUM, 1, D), lambda: (0, 0, 0)),
            pl.BlockSpec((LNUM, 1), lambda: (0, 0)),
            pl.BlockSpec((B, 1), lambda: (0, 0)),
            pl.BlockSpec(memory_space=pltpu.SMEM),
        ],
        out_specs=[
            pl.BlockSpec((1, 1), lambda: (0, 0)),
            pl.BlockSpec((B, 1), lambda: (0, 0)),
        ],
        out_shape=[
            jax.ShapeDtypeStruct((1, 1), jnp.float32),
            jax.ShapeDtypeStruct((B, 1), jnp.float32),
        ],
    )(hs0, hs1, h20, h21, W2, Wa, Wlr, blr, targets, lp)


def kernel(nodes, targets, layer_predict, neighbors, feat_tables, W1, W2,
           Wa, Wlr, blr):
    nodes = nodes.astype(jnp.int32)
    neighbors = neighbors.astype(jnp.int32)
    nbpad = jnp.concatenate(
        [neighbors, jnp.zeros((LNUM, N, 16 - S), jnp.int32)], axis=2)
    pk0 = nbpad[0].reshape(N // 8, 128)
    pk1 = nbpad[1].reshape(N // 8, 128)
    g0, g1 = _g_precompute(feat_tables, W1)
    hs0, hs1, h20, h21 = _sc_aggregate(nodes, pk0, pk1, g0, g1)
    lp = jnp.asarray(layer_predict, jnp.int32).reshape(1)
    loss2, pred = _finalize(hs0, hs1, h20, h21, W2, Wa, Wlr, blr,
                            targets.astype(jnp.float32), lp)
    return loss2[0, 0], pred


# speedup vs baseline: 8.0227x; 1.3981x over previous
import jax
import jax.numpy as jnp
import numpy as np
from jax import lax
from jax.experimental import pallas as pl
from jax.experimental.pallas import tpu as pltpu
from jax.experimental.pallas import tpu_sc as plsc

LNUM = 2
N = 50000
F = 128
H = 128
D = 128
B = 4096
S = 10

NC, NS = 2, 16
NW = NC * NS
BPW = B // NW
W1S = BPW * S
INV = 1.0 / (S + 1.0)
_CS = 16


_GBLK = 1000


def _g_body(feat_ref, w1_ref, g0_ref, g1_ref):
    dn = (((1,), (1,)), ((), ()))
    g0_ref[...] = lax.dot_general(
        feat_ref[0], w1_ref[0], dn, precision=lax.Precision.HIGHEST,
        preferred_element_type=jnp.float32) * INV
    g1_ref[...] = lax.dot_general(
        feat_ref[1], w1_ref[1], dn, precision=lax.Precision.HIGHEST,
        preferred_element_type=jnp.float32) * INV


def _g_precompute(feat_tables, W1):
    return pl.pallas_call(
        _g_body,
        grid=(N // _GBLK,),
        in_specs=[
            pl.BlockSpec((LNUM, _GBLK, F), lambda i: (0, i, 0)),
            pl.BlockSpec((LNUM, H, F), lambda i: (0, 0, 0)),
        ],
        out_specs=[
            pl.BlockSpec((_GBLK, H), lambda i: (i, 0)),
            pl.BlockSpec((_GBLK, H), lambda i: (i, 0)),
        ],
        out_shape=[
            jax.ShapeDtypeStruct((N, H), jnp.float32),
            jax.ShapeDtypeStruct((N, H), jnp.float32),
        ],
    )(feat_tables, W1)


def _sum_chunk_item(selfg, nbg, i):
    out = []
    for v in range(H // 16):
        t = selfg[i, pl.ds(v * 16, 16)]
        for q in range(S):
            t = t + nbg[q * _CS + i, pl.ds(v * 16, 16)]
        out.append(jnp.maximum(t, 0.0))
    return out


def _zero_buf(buf):
    z = jnp.zeros((16,), jnp.float32)

    def zrow(i, c):
        for v in range(H // 16):
            buf[i, pl.ds(v * 16, 16)] = z
        return c

    lax.fori_loop(0, BPW, zrow, 0)


def _g_chunk_dmas(gt, self_idx_src, self_off, nb_idx_src, nb_base,
                  selfg, nbg, sem):
    soff = pl.multiple_of(self_off, _CS)
    ds = [pltpu.make_async_copy(
        gt.at[self_idx_src.at[pl.ds(soff, _CS)]], selfg, sem)]
    for s in range(S):
        off = pl.multiple_of(nb_base(s), _CS)
        ds.append(pltpu.make_async_copy(
            gt.at[nb_idx_src.at[pl.ds(off, _CS)]],
            nbg.at[pl.ds(s * _CS, _CS)], sem))
    return ds


def _extract_ids(src_ids, idrows, rowidx, dst, dst_base, dst_stride):
    iota = lax.broadcasted_iota(jnp.int32, (16,), 0)

    def chunk(t, c):
        toff = pl.multiple_of(t * 16, 16)
        v = src_ids[pl.ds(toff, 16)]
        colb = (v & 7) * 16
        rows = iota + toff
        for s in range(S):
            ids = plsc.load_gather(idrows, [rows, colb + s])
            off = pl.multiple_of(dst_base + s * dst_stride + toff, 16)
            dst[pl.ds(off, 16)] = ids
        return c

    lax.fori_loop(0, BPW // 16, chunk, 0)


def _fill_rowidx(src_ids, src_off, rowidx):
    def chunk(t, c):
        toff = pl.multiple_of(t * 16, 16)
        v = src_ids[pl.ds(src_off + toff, 16)]
        rowidx[pl.ds(toff, 16)] = lax.shift_right_logical(v, 3)
        return c

    lax.fori_loop(0, BPW // 16, chunk, 0)


def _pipelined_phase(gt, nch, self_src, self_off_fn, nb_src, nb_base_fn,
                     selfg2, nbg2, sem2, compute_fn):
    def dmas(c, p):
        return _g_chunk_dmas(
            gt, self_src, self_off_fn(c), nb_src,
            lambda s, c=c: nb_base_fn(c, s),
            selfg2.at[p], nbg2.at[p], sem2.at[p])

    for d in dmas(0, 0):
        d.start()

    def step(c, carry):
        p = lax.rem(c, 2)
        for d in dmas(c, p):
            d.wait()

        @pl.when(c + 1 < nch)
        def _():
            for d in dmas(c + 1, 1 - p):
                d.start()

        compute_fn(c, selfg2.at[p], nbg2.at[p])
        return carry

    lax.fori_loop(0, nch, step, 0)


def _sc_body(nodes_hbm, pk0, pk1, g0, g1, hs0, hs1, h20, h21,
             nodes_v, nb1w, nb2w, idrows, rowidx, selfg, nbg, hbuf, sem):
    wid = lax.axis_index("s") * NC + lax.axis_index("c")
    base = pl.multiple_of(wid * BPW, BPW)
    pltpu.sync_copy(nodes_hbm.at[pl.ds(base, BPW)], nodes_v)

    for pk, gt, hs_out, h2_out in ((pk0, g0, hs0, h20),
                                   (pk1, g1, hs1, h21)):
        _fill_rowidx(nodes_v, 0, rowidx)
        pltpu.sync_copy(pk.at[rowidx], idrows)
        _extract_ids(nodes_v, idrows, rowidx, nb1w, 0, BPW)

        def nb2_step(k, c, pk=pk):
            koff = pl.multiple_of(k * BPW, BPW)
            _fill_rowidx(nb1w, koff, rowidx)
            pltpu.sync_copy(pk.at[rowidx], idrows)
            _extract_ids(nb1w.at[pl.ds(koff, BPW)], idrows, rowidx,
                         nb2w, koff, W1S)
            return c

        lax.fori_loop(0, W1S // BPW, nb2_step, 0)

        def self_compute(c, sg, ng):
            def item(i, cc, c=c):
                vals = _sum_chunk_item(sg, ng, i)
                row = c * _CS + i
                for v in range(H // 16):
                    hbuf[row, pl.ds(v * 16, 16)] = vals[v]
                return cc

            lax.fori_loop(0, _CS, item, 0)

        _pipelined_phase(
            gt, BPW // _CS,
            nodes_v, lambda c: c * _CS,
            nb1w, lambda c, s: s * BPW + c * _CS,
            selfg, nbg, sem, self_compute)
        pltpu.sync_copy(hbuf, hs_out.at[pl.ds(base, BPW)])

        _zero_buf(hbuf)

        def nb_compute(c, sg, ng):
            def item(i, cc, c=c):
                vals = _sum_chunk_item(sg, ng, i)
                brow = lax.rem(c, BPW // _CS) * _CS + i
                for v in range(H // 16):
                    sl = pl.ds(v * 16, 16)
                    hbuf[brow, sl] = hbuf[brow, sl] + vals[v]
                return cc

            lax.fori_loop(0, _CS, item, 0)

        _pipelined_phase(
            gt, W1S // _CS,
            nb1w, lambda c: c * _CS,
            nb2w, lambda c, s: s * W1S + c * _CS,
            selfg, nbg, sem, nb_compute)
        pltpu.sync_copy(hbuf, h2_out.at[pl.ds(base, BPW)])


def _sc_aggregate(nodes, pk0, pk1, g0, g1):
    mesh = plsc.VectorSubcoreMesh(
        core_axis_name="c", subcore_axis_name="s",
        num_cores=NC, num_subcores=NS)
    out = jax.ShapeDtypeStruct((B, H), jnp.float32)
    f = pl.kernel(
        _sc_body,
        out_type=(out, out, out, out),
        mesh=mesh,
        scratch_types=[
            pltpu.VMEM((BPW,), jnp.int32),
            pltpu.VMEM((W1S,), jnp.int32),
            pltpu.VMEM((W1S * S,), jnp.int32),
            pltpu.VMEM((BPW, 128), jnp.int32),
            pltpu.VMEM((BPW,), jnp.int32),
            pltpu.VMEM((2, _CS, H), jnp.float32),
            pltpu.VMEM((2, S * _CS, H), jnp.float32),
            pltpu.VMEM((BPW, H), jnp.float32),
            pltpu.SemaphoreType.DMA((2,)),
        ],
        compiler_params=pltpu.CompilerParams(needs_layout_passes=False),
    )
    return f(nodes, pk0, pk1, g0, g1)


def _fin_body(hs0, hs1, h20, h21, w2_ref, wa_ref, wlr_ref, blr_ref,
              tgt_ref, lp_ref, loss_ref, pred_ref):
    dn_t = (((1,), (1,)), ((), ()))
    dn = (((1,), (0,)), ((), ()))
    hp = lax.Precision.HIGHEST
    lp = lp_ref[0]

    embs = []
    for l, (hs, h2) in enumerate(((hs0, h20), (hs1, h21))):
        agg2 = (h2[...] + hs[...]) * INV
        emb = jnp.maximum(
            lax.dot_general(agg2, w2_ref[l], dn_t, precision=hp,
                            preferred_element_type=jnp.float32), 0.0)
        embs.append(emb)

    q = jnp.where(lp == 0, embs[0], embs[1])
    tq = jnp.tanh(q)
    scale = 1.0 / np.sqrt(D)
    scores = []
    for l in range(LNUM):
        proj = lax.dot_general(embs[l], wa_ref[...], dn, precision=hp,
                               preferred_element_type=jnp.float32)
        scores.append(
            jnp.sum(tq * jnp.tanh(proj), axis=1, keepdims=True) * scale)

    m = jnp.maximum(scores[0], scores[1])
    e0 = jnp.exp(scores[0] - m)
    e1 = jnp.exp(scores[1] - m)
    denom = e0 + e1
    fused = (e0 * embs[0] + e1 * embs[1]) / denom

    wlr = jnp.where(lp == 0, wlr_ref[0], wlr_ref[1])
    blr = jnp.where(lp == 0, blr_ref[0, 0], blr_ref[1, 0])
    logits = jnp.sum(fused * wlr, axis=1, keepdims=True) + blr
    pred = 1.0 / (1.0 + jnp.exp(-logits))
    pred_ref[...] = pred

    eps = 1e-7
    p = jnp.clip(pred, eps, 1.0 - eps)
    t = tgt_ref[...]
    ll = t * jnp.log(p) + (1.0 - t) * jnp.log(1.0 - p)
    loss_ref[...] = (-jnp.sum(ll) * (1.0 / B)).reshape(1, 1)


def _finalize(hs0, hs1, h20, h21, W2, Wa, Wlr, blr, targets, lp):
    return pl.pallas_call(
        _fin_body,
        in_specs=[pl.BlockSpec((B, H), lambda: (0, 0))] * 4 + [
            pl.BlockSpec((LNUM, D, H), lambda: (0, 0, 0)),
            pl.BlockSpec((D, D), lambda: (0, 0)),
            pl.BlockSpec((LNUM, 1, D), lambda: (0, 0, 0)),
            pl.BlockSpec((LNUM, 1), lambda: (0, 0)),
            pl.BlockSpec((B, 1), lambda: (0, 0)),
            pl.BlockSpec(memory_space=pltpu.SMEM),
        ],
        out_specs=[
            pl.BlockSpec((1, 1), lambda: (0, 0)),
            pl.BlockSpec((B, 1), lambda: (0, 0)),
        ],
        out_shape=[
            jax.ShapeDtypeStruct((1, 1), jnp.float32),
            jax.ShapeDtypeStruct((B, 1), jnp.float32),
        ],
    )(hs0, hs1, h20, h21, W2, Wa, Wlr, blr, targets, lp)


def kernel(nodes, targets, layer_predict, neighbors, feat_tables, W1, W2,
           Wa, Wlr, blr):
    nodes = nodes.astype(jnp.int32)
    neighbors = neighbors.astype(jnp.int32)
    nbpad = jnp.concatenate(
        [neighbors, jnp.zeros((LNUM, N, 16 - S), jnp.int32)], axis=2)
    pk0 = nbpad[0].reshape(N // 8, 128)
    pk1 = nbpad[1].reshape(N // 8, 128)
    g0, g1 = _g_precompute(feat_tables, W1)
    hs0, hs1, h20, h21 = _sc_aggregate(nodes, pk0, pk1, g0, g1)
    lp = jnp.asarray(layer_predict, jnp.int32).reshape(1)
    loss2, pred = _finalize(hs0, hs1, h20, h21, W2, Wa, Wlr, blr,
                            targets.astype(jnp.float32), lp)
    return loss2[0, 0], pred


# speedup vs baseline: 9.0623x; 1.1296x over previous
import jax
import jax.numpy as jnp
import numpy as np
from jax import lax
from jax.experimental import pallas as pl
from jax.experimental.pallas import tpu as pltpu
from jax.experimental.pallas import tpu_sc as plsc

LNUM = 2
N = 50000
F = 128
H = 128
D = 128
B = 4096
S = 10

NC, NS = 2, 16
NW = NC * NS
BPW = B // NW
W1S = BPW * S
INV = 1.0 / (S + 1.0)
_CS = 32


_GBLK = 1000


def _g_body(feat_ref, w1_ref, g0_ref, g1_ref):
    dn = (((1,), (1,)), ((), ()))
    g0_ref[...] = lax.dot_general(
        feat_ref[0], w1_ref[0], dn, precision=lax.Precision.HIGHEST,
        preferred_element_type=jnp.float32) * INV
    g1_ref[...] = lax.dot_general(
        feat_ref[1], w1_ref[1], dn, precision=lax.Precision.HIGHEST,
        preferred_element_type=jnp.float32) * INV


def _g_precompute(feat_tables, W1):
    return pl.pallas_call(
        _g_body,
        grid=(N // _GBLK,),
        in_specs=[
            pl.BlockSpec((LNUM, _GBLK, F), lambda i: (0, i, 0)),
            pl.BlockSpec((LNUM, H, F), lambda i: (0, 0, 0)),
        ],
        out_specs=[
            pl.BlockSpec((_GBLK, H), lambda i: (i, 0)),
            pl.BlockSpec((_GBLK, H), lambda i: (i, 0)),
        ],
        out_shape=[
            jax.ShapeDtypeStruct((N, H), jnp.float32),
            jax.ShapeDtypeStruct((N, H), jnp.float32),
        ],
    )(feat_tables, W1)


def _sum_chunk_item(selfg, nbg, i):
    out = []
    for v in range(H // 16):
        t = selfg[i, pl.ds(v * 16, 16)]
        for q in range(S):
            t = t + nbg[q * _CS + i, pl.ds(v * 16, 16)]
        out.append(jnp.maximum(t, 0.0))
    return out


def _zero_buf(buf):
    z = jnp.zeros((16,), jnp.float32)

    def zrow(i, c):
        for v in range(H // 16):
            buf[i, pl.ds(v * 16, 16)] = z
        return c

    lax.fori_loop(0, BPW, zrow, 0)


def _g_chunk_dmas(gt, self_idx_src, self_off, nb_idx_src, nb_base,
                  selfg, nbg, sem):
    soff = pl.multiple_of(self_off, _CS)
    ds = [pltpu.make_async_copy(
        gt.at[self_idx_src.at[pl.ds(soff, _CS)]], selfg, sem)]
    for s in range(S):
        off = pl.multiple_of(nb_base(s), _CS)
        ds.append(pltpu.make_async_copy(
            gt.at[nb_idx_src.at[pl.ds(off, _CS)]],
            nbg.at[pl.ds(s * _CS, _CS)], sem))
    return ds


def _extract_ids(src_ids, idrows, rowidx, dst, dst_base, dst_stride):
    iota = lax.broadcasted_iota(jnp.int32, (16,), 0)

    def chunk(t, c):
        toff = pl.multiple_of(t * 16, 16)
        v = src_ids[pl.ds(toff, 16)]
        colb = (v & 7) * 16
        rows = iota + toff
        for s in range(S):
            ids = plsc.load_gather(idrows, [rows, colb + s])
            if ids.dtype != jnp.int32:
                ids = plsc.bitcast(ids, jnp.int32)
            off = pl.multiple_of(dst_base + s * dst_stride + toff, 16)
            dst[pl.ds(off, 16)] = ids
        return c

    lax.fori_loop(0, BPW // 16, chunk, 0)


def _fill_rowidx(src_ids, src_off, rowidx):
    def chunk(t, c):
        toff = pl.multiple_of(t * 16, 16)
        v = src_ids[pl.ds(src_off + toff, 16)]
        rowidx[pl.ds(toff, 16)] = lax.shift_right_logical(v, 3)
        return c

    lax.fori_loop(0, BPW // 16, chunk, 0)


def _pipelined_phase(gt, nch, self_src, self_off_fn, nb_src, nb_base_fn,
                     selfg2, nbg2, sem2, compute_fn):
    def dmas(c, p):
        return _g_chunk_dmas(
            gt, self_src, self_off_fn(c), nb_src,
            lambda s, c=c: nb_base_fn(c, s),
            selfg2.at[p], nbg2.at[p], sem2.at[p])

    for d in dmas(0, 0):
        d.start()

    def step(c, carry):
        p = lax.rem(c, 2)
        for d in dmas(c, p):
            d.wait()

        @pl.when(c + 1 < nch)
        def _():
            for d in dmas(c + 1, 1 - p):
                d.start()

        compute_fn(c, selfg2.at[p], nbg2.at[p])
        return carry

    lax.fori_loop(0, nch, step, 0)


def _sc_body(nodes_hbm, pk0, pk1, g0, g1, hs0, hs1, h20, h21,
             nodes_v, nb1w, nb2w, rowidx, selfg, nbg, hbuf, sem):
    wid = lax.axis_index("s") * NC + lax.axis_index("c")
    base = pl.multiple_of(wid * BPW, BPW)
    pltpu.sync_copy(nodes_hbm.at[pl.ds(base, BPW)], nodes_v)
    idrows_dma = nbg.at[0, pl.ds(0, BPW)].bitcast(jnp.int32)
    idrows = nbg.at[0, pl.ds(0, BPW)]

    for pk, gt, hs_out, h2_out in ((pk0, g0, hs0, h20),
                                   (pk1, g1, hs1, h21)):
        _fill_rowidx(nodes_v, 0, rowidx)
        pltpu.sync_copy(pk.at[rowidx], idrows_dma)
        _extract_ids(nodes_v, idrows, rowidx, nb1w, 0, BPW)

        def nb2_step(k, c, pk=pk):
            koff = pl.multiple_of(k * BPW, BPW)
            _fill_rowidx(nb1w, koff, rowidx)
            pltpu.sync_copy(pk.at[rowidx], idrows_dma)
            _extract_ids(nb1w.at[pl.ds(koff, BPW)], idrows, rowidx,
                         nb2w, koff, W1S)
            return c

        lax.fori_loop(0, W1S // BPW, nb2_step, 0)

        def self_compute(c, sg, ng):
            def item(i, cc, c=c):
                vals = _sum_chunk_item(sg, ng, i)
                row = c * _CS + i
                for v in range(H // 16):
                    hbuf[row, pl.ds(v * 16, 16)] = vals[v]
                return cc

            lax.fori_loop(0, _CS, item, 0)

        _pipelined_phase(
            gt, BPW // _CS,
            nodes_v, lambda c: c * _CS,
            nb1w, lambda c, s: s * BPW + c * _CS,
            selfg, nbg, sem, self_compute)
        pltpu.sync_copy(hbuf, hs_out.at[pl.ds(base, BPW)])

        _zero_buf(hbuf)

        def nb_compute(c, sg, ng):
            def item(i, cc, c=c):
                vals = _sum_chunk_item(sg, ng, i)
                brow = lax.rem(c, BPW // _CS) * _CS + i
                for v in range(H // 16):
                    plsc.addupdate(hbuf.at[brow, pl.ds(v * 16, 16)], vals[v])
                return cc

            lax.fori_loop(0, _CS, item, 0)

        _pipelined_phase(
            gt, W1S // _CS,
            nb1w, lambda c: c * _CS,
            nb2w, lambda c, s: s * W1S + c * _CS,
            selfg, nbg, sem, nb_compute)
        pltpu.sync_copy(hbuf, h2_out.at[pl.ds(base, BPW)])


def _sc_aggregate(nodes, pk0, pk1, g0, g1):
    mesh = plsc.VectorSubcoreMesh(
        core_axis_name="c", subcore_axis_name="s",
        num_cores=NC, num_subcores=NS)
    out = jax.ShapeDtypeStruct((B, H), jnp.float32)
    f = pl.kernel(
        _sc_body,
        out_type=(out, out, out, out),
        mesh=mesh,
        scratch_types=[
            pltpu.VMEM((BPW,), jnp.int32),
            pltpu.VMEM((W1S,), jnp.int32),
            pltpu.VMEM((W1S * S,), jnp.int32),
            pltpu.VMEM((BPW,), jnp.int32),
            pltpu.VMEM((2, _CS, H), jnp.float32),
            pltpu.VMEM((2, S * _CS, H), jnp.float32),
            pltpu.VMEM((BPW, H), jnp.float32),
            pltpu.SemaphoreType.DMA((2,)),
        ],
        compiler_params=pltpu.CompilerParams(needs_layout_passes=False),
    )
    return f(nodes, pk0, pk1, g0, g1)


def _fin_body(hs0, hs1, h20, h21, w2_ref, wa_ref, wlr_ref, blr_ref,
              tgt_ref, lp_ref, loss_ref, pred_ref):
    dn_t = (((1,), (1,)), ((), ()))
    dn = (((1,), (0,)), ((), ()))
    hp = lax.Precision.HIGHEST
    lp = lp_ref[0]

    embs = []
    for l, (hs, h2) in enumerate(((hs0, h20), (hs1, h21))):
        agg2 = (h2[...] + hs[...]) * INV
        emb = jnp.maximum(
            lax.dot_general(agg2, w2_ref[l], dn_t, precision=hp,
                            preferred_element_type=jnp.float32), 0.0)
        embs.append(emb)

    q = jnp.where(lp == 0, embs[0], embs[1])
    tq = jnp.tanh(q)
    scale = 1.0 / np.sqrt(D)
    scores = []
    for l in range(LNUM):
        proj = lax.dot_general(embs[l], wa_ref[...], dn, precision=hp,
                               preferred_element_type=jnp.float32)
        scores.append(
            jnp.sum(tq * jnp.tanh(proj), axis=1, keepdims=True) * scale)

    m = jnp.maximum(scores[0], scores[1])
    e0 = jnp.exp(scores[0] - m)
    e1 = jnp.exp(scores[1] - m)
    denom = e0 + e1
    fused = (e0 * embs[0] + e1 * embs[1]) / denom

    wlr = jnp.where(lp == 0, wlr_ref[0], wlr_ref[1])
    blr = jnp.where(lp == 0, blr_ref[0, 0], blr_ref[1, 0])
    logits = jnp.sum(fused * wlr, axis=1, keepdims=True) + blr
    pred = 1.0 / (1.0 + jnp.exp(-logits))
    pred_ref[...] = pred

    eps = 1e-7
    p = jnp.clip(pred, eps, 1.0 - eps)
    t = tgt_ref[...]
    ll = t * jnp.log(p) + (1.0 - t) * jnp.log(1.0 - p)
    loss_ref[...] = (-jnp.sum(ll) * (1.0 / B)).reshape(1, 1)


def _finalize(hs0, hs1, h20, h21, W2, Wa, Wlr, blr, targets, lp):
    return pl.pallas_call(
        _fin_body,
        in_specs=[pl.BlockSpec((B, H), lambda: (0, 0))] * 4 + [
            pl.BlockSpec((LNUM, D, H), lambda: (0, 0, 0)),
            pl.BlockSpec((D, D), lambda: (0, 0)),
            pl.BlockSpec((LNUM, 1, D), lambda: (0, 0, 0)),
            pl.BlockSpec((LNUM, 1), lambda: (0, 0)),
            pl.BlockSpec((B, 1), lambda: (0, 0)),
            pl.BlockSpec(memory_space=pltpu.SMEM),
        ],
        out_specs=[
            pl.BlockSpec((1, 1), lambda: (0, 0)),
            pl.BlockSpec((B, 1), lambda: (0, 0)),
        ],
        out_shape=[
            jax.ShapeDtypeStruct((1, 1), jnp.float32),
            jax.ShapeDtypeStruct((B, 1), jnp.float32),
        ],
    )(hs0, hs1, h20, h21, W2, Wa, Wlr, blr, targets, lp)


def kernel(nodes, targets, layer_predict, neighbors, feat_tables, W1, W2,
           Wa, Wlr, blr):
    nodes = nodes.astype(jnp.int32)
    neighbors = neighbors.astype(jnp.int32)
    nbpad = jnp.concatenate(
        [neighbors, jnp.zeros((LNUM, N, 16 - S), jnp.int32)], axis=2)
    pk0 = nbpad[0].reshape(N // 8, 128)
    pk1 = nbpad[1].reshape(N // 8, 128)
    g0, g1 = _g_precompute(feat_tables, W1)
    hs0, hs1, h20, h21 = _sc_aggregate(nodes, pk0, pk1, g0, g1)
    lp = jnp.asarray(layer_predict, jnp.int32).reshape(1)
    loss2, pred = _finalize(hs0, hs1, h20, h21, W2, Wa, Wlr, blr,
                            targets.astype(jnp.float32), lp)
    return loss2[0, 0], pred


# speedup vs baseline: 9.3790x; 1.0349x over previous
import jax
import jax.numpy as jnp
import numpy as np
from jax import lax
from jax.experimental import pallas as pl
from jax.experimental.pallas import tpu as pltpu
from jax.experimental.pallas import tpu_sc as plsc

LNUM = 2
N = 50000
F = 128
H = 128
D = 128
B = 4096
S = 10

NC, NS = 2, 16
NW = NC * NS
BPW = B // NW
W1S = BPW * S
INV = 1.0 / (S + 1.0)
_CS = 32


_GBLK = 1000


def _g_body(feat_ref, w1_ref, g0_ref, g1_ref):
    dn = (((1,), (1,)), ((), ()))
    g0_ref[...] = lax.dot_general(
        feat_ref[0], w1_ref[0], dn, precision=lax.Precision.DEFAULT,
        preferred_element_type=jnp.float32) * INV
    g1_ref[...] = lax.dot_general(
        feat_ref[1], w1_ref[1], dn, precision=lax.Precision.DEFAULT,
        preferred_element_type=jnp.float32) * INV


def _pack_tables(neighbors):
    nbpad = jnp.concatenate(
        [neighbors, jnp.zeros((LNUM, N, 16 - S), jnp.int32)], axis=2)
    return nbpad.reshape(LNUM, N // 8, 128)


def _g_precompute(feat_tables, W1):
    return pl.pallas_call(
        _g_body,
        grid=(N // _GBLK,),
        in_specs=[
            pl.BlockSpec((LNUM, _GBLK, F), lambda i: (0, i, 0)),
            pl.BlockSpec((LNUM, H, F), lambda i: (0, 0, 0)),
        ],
        out_specs=[
            pl.BlockSpec((_GBLK, H), lambda i: (i, 0)),
            pl.BlockSpec((_GBLK, H), lambda i: (i, 0)),
        ],
        out_shape=[
            jax.ShapeDtypeStruct((N, H), jnp.float32),
            jax.ShapeDtypeStruct((N, H), jnp.float32),
        ],
    )(feat_tables, W1)


def _sum_chunk_item(selfg, nbg, i):
    out = []
    for v in range(H // 16):
        t = selfg[i, pl.ds(v * 16, 16)]
        for q in range(S):
            t = t + nbg[q * _CS + i, pl.ds(v * 16, 16)]
        out.append(jnp.maximum(t, 0.0))
    return out


def _zero_buf(buf):
    z = jnp.zeros((16,), jnp.float32)

    def zrow(i, c):
        for v in range(H // 16):
            buf[i, pl.ds(v * 16, 16)] = z
        return c

    lax.fori_loop(0, BPW, zrow, 0)


def _g_chunk_dmas(gt, self_idx_src, self_off, nb_idx_src, nb_base,
                  selfg, nbg, sem):
    soff = pl.multiple_of(self_off, _CS)
    ds = [pltpu.make_async_copy(
        gt.at[self_idx_src.at[pl.ds(soff, _CS)]], selfg, sem)]
    for s in range(S):
        off = pl.multiple_of(nb_base(s), _CS)
        ds.append(pltpu.make_async_copy(
            gt.at[nb_idx_src.at[pl.ds(off, _CS)]],
            nbg.at[pl.ds(s * _CS, _CS)], sem))
    return ds


def _extract_ids(src_ids, idrows, rowidx, dst, dst_base, dst_stride):
    iota = lax.broadcasted_iota(jnp.int32, (16,), 0)

    def chunk(t, c):
        toff = pl.multiple_of(t * 16, 16)
        v = src_ids[pl.ds(toff, 16)]
        colb = (v & 7) * 16
        rows = iota + toff
        for s in range(S):
            ids = plsc.load_gather(idrows, [rows, colb + s])
            if ids.dtype != jnp.int32:
                ids = plsc.bitcast(ids, jnp.int32)
            off = pl.multiple_of(dst_base + s * dst_stride + toff, 16)
            dst[pl.ds(off, 16)] = ids
        return c

    lax.fori_loop(0, BPW // 16, chunk, 0)


def _fill_rowidx(src_ids, src_off, rowidx):
    def chunk(t, c):
        toff = pl.multiple_of(t * 16, 16)
        v = src_ids[pl.ds(src_off + toff, 16)]
        rowidx[pl.ds(toff, 16)] = lax.shift_right_logical(v, 3)
        return c

    lax.fori_loop(0, BPW // 16, chunk, 0)


def _pipelined_phase(gt, nch, self_src, self_off_fn, nb_src, nb_base_fn,
                     selfg2, nbg2, sem2, compute_fn):
    def dmas(c, p):
        return _g_chunk_dmas(
            gt, self_src, self_off_fn(c), nb_src,
            lambda s, c=c: nb_base_fn(c, s),
            selfg2.at[p], nbg2.at[p], sem2.at[p])

    for d in dmas(0, 0):
        d.start()

    def step(c, carry):
        p = lax.rem(c, 2)
        for d in dmas(c, p):
            d.wait()

        @pl.when(c + 1 < nch)
        def _():
            for d in dmas(c + 1, 1 - p):
                d.start()

        compute_fn(c, selfg2.at[p], nbg2.at[p])
        return carry

    lax.fori_loop(0, nch, step, 0)


def _sc_body(nodes_hbm, pkt, g0, g1, hs0, hs1, h20, h21,
             nodes_v, nb1w, nb2w, rowidx, selfg, nbg, hbuf, sem):
    wid = lax.axis_index("s") * NC + lax.axis_index("c")
    base = pl.multiple_of(wid * BPW, BPW)
    pltpu.sync_copy(nodes_hbm.at[pl.ds(base, BPW)], nodes_v)
    idrows_dma = nbg.at[0, pl.ds(0, BPW)].bitcast(jnp.int32)
    idrows = nbg.at[0, pl.ds(0, BPW)]

    for pk, gt, hs_out, h2_out in ((pkt.at[0], g0, hs0, h20),
                                   (pkt.at[1], g1, hs1, h21)):
        _fill_rowidx(nodes_v, 0, rowidx)
        pltpu.sync_copy(pk.at[rowidx], idrows_dma)
        _extract_ids(nodes_v, idrows, rowidx, nb1w, 0, BPW)

        def nb2_step(k, c, pk=pk):
            koff = pl.multiple_of(k * BPW, BPW)
            _fill_rowidx(nb1w, koff, rowidx)
            pltpu.sync_copy(pk.at[rowidx], idrows_dma)
            _extract_ids(nb1w.at[pl.ds(koff, BPW)], idrows, rowidx,
                         nb2w, koff, W1S)
            return c

        lax.fori_loop(0, W1S // BPW, nb2_step, 0)

        def self_compute(c, sg, ng):
            def item(i, cc, c=c):
                vals = _sum_chunk_item(sg, ng, i)
                row = c * _CS + i
                for v in range(H // 16):
                    hbuf[row, pl.ds(v * 16, 16)] = vals[v]
                return cc

            lax.fori_loop(0, _CS, item, 0)

        _pipelined_phase(
            gt, BPW // _CS,
            nodes_v, lambda c: c * _CS,
            nb1w, lambda c, s: s * BPW + c * _CS,
            selfg, nbg, sem, self_compute)
        pltpu.sync_copy(hbuf, hs_out.at[pl.ds(base, BPW)])

        _zero_buf(hbuf)

        def nb_compute(c, sg, ng):
            def item(i, cc, c=c):
                vals = _sum_chunk_item(sg, ng, i)
                brow = lax.rem(c, BPW // _CS) * _CS + i
                for v in range(H // 16):
                    plsc.addupdate(hbuf.at[brow, pl.ds(v * 16, 16)], vals[v])
                return cc

            lax.fori_loop(0, _CS, item, 0)

        _pipelined_phase(
            gt, W1S // _CS,
            nb1w, lambda c: c * _CS,
            nb2w, lambda c, s: s * W1S + c * _CS,
            selfg, nbg, sem, nb_compute)
        pltpu.sync_copy(hbuf, h2_out.at[pl.ds(base, BPW)])


def _sc_aggregate(nodes, pkt, g0, g1):
    mesh = plsc.VectorSubcoreMesh(
        core_axis_name="c", subcore_axis_name="s",
        num_cores=NC, num_subcores=NS)
    out = jax.ShapeDtypeStruct((B, H), jnp.float32)
    f = pl.kernel(
        _sc_body,
        out_type=(out, out, out, out),
        mesh=mesh,
        scratch_types=[
            pltpu.VMEM((BPW,), jnp.int32),
            pltpu.VMEM((W1S,), jnp.int32),
            pltpu.VMEM((W1S * S,), jnp.int32),
            pltpu.VMEM((BPW,), jnp.int32),
            pltpu.VMEM((2, _CS, H), jnp.float32),
            pltpu.VMEM((2, S * _CS, H), jnp.float32),
            pltpu.VMEM((BPW, H), jnp.float32),
            pltpu.SemaphoreType.DMA((2,)),
        ],
        compiler_params=pltpu.CompilerParams(needs_layout_passes=False),
    )
    return f(nodes, pkt, g0, g1)


def _fin_body(hs0, hs1, h20, h21, w2_ref, wa_ref, wlr_ref, blr_ref,
              tgt_ref, lp_ref, loss_ref, pred_ref):
    dn_t = (((1,), (1,)), ((), ()))
    dn = (((1,), (0,)), ((), ()))
    hp = lax.Precision.HIGHEST
    lp = lp_ref[0]

    embs = []
    for l, (hs, h2) in enumerate(((hs0, h20), (hs1, h21))):
        agg2 = (h2[...] + hs[...]) * INV
        emb = jnp.maximum(
            lax.dot_general(agg2, w2_ref[l], dn_t, precision=hp,
                            preferred_element_type=jnp.float32), 0.0)
        embs.append(emb)

    q = jnp.where(lp == 0, embs[0], embs[1])
    tq = jnp.tanh(q)
    scale = 1.0 / np.sqrt(D)
    scores = []
    for l in range(LNUM):
        proj = lax.dot_general(embs[l], wa_ref[...], dn, precision=hp,
                               preferred_element_type=jnp.float32)
        scores.append(
            jnp.sum(tq * jnp.tanh(proj), axis=1, keepdims=True) * scale)

    m = jnp.maximum(scores[0], scores[1])
    e0 = jnp.exp(scores[0] - m)
    e1 = jnp.exp(scores[1] - m)
    denom = e0 + e1
    fused = (e0 * embs[0] + e1 * embs[1]) / denom

    wlr = jnp.where(lp == 0, wlr_ref[0], wlr_ref[1])
    blr = jnp.where(lp == 0, blr_ref[0, 0], blr_ref[1, 0])
    logits = jnp.sum(fused * wlr, axis=1, keepdims=True) + blr
    pred = 1.0 / (1.0 + jnp.exp(-logits))
    pred_ref[...] = pred

    eps = 1e-7
    p = jnp.clip(pred, eps, 1.0 - eps)
    t = tgt_ref[...]
    ll = t * jnp.log(p) + (1.0 - t) * jnp.log(1.0 - p)
    loss_ref[...] = (-jnp.sum(ll) * (1.0 / B)).reshape(1, 1)


def _finalize(hs0, hs1, h20, h21, W2, Wa, Wlr, blr, targets, lp):
    return pl.pallas_call(
        _fin_body,
        in_specs=[pl.BlockSpec((B, H), lambda: (0, 0))] * 4 + [
            pl.BlockSpec((LNUM, D, H), lambda: (0, 0, 0)),
            pl.BlockSpec((D, D), lambda: (0, 0)),
            pl.BlockSpec((LNUM, 1, D), lambda: (0, 0, 0)),
            pl.BlockSpec((LNUM, 1), lambda: (0, 0)),
            pl.BlockSpec((B, 1), lambda: (0, 0)),
            pl.BlockSpec(memory_space=pltpu.SMEM),
        ],
        out_specs=[
            pl.BlockSpec((1, 1), lambda: (0, 0)),
            pl.BlockSpec((B, 1), lambda: (0, 0)),
        ],
        out_shape=[
            jax.ShapeDtypeStruct((1, 1), jnp.float32),
            jax.ShapeDtypeStruct((B, 1), jnp.float32),
        ],
    )(hs0, hs1, h20, h21, W2, Wa, Wlr, blr, targets, lp)


def kernel(nodes, targets, layer_predict, neighbors, feat_tables, W1, W2,
           Wa, Wlr, blr):
    nodes = nodes.astype(jnp.int32)
    neighbors = neighbors.astype(jnp.int32)
    pkt = _pack_tables(neighbors)
    g0, g1 = _g_precompute(feat_tables, W1)
    hs0, hs1, h20, h21 = _sc_aggregate(nodes, pkt, g0, g1)
    lp = jnp.asarray(layer_predict, jnp.int32).reshape(1)
    loss2, pred = _finalize(hs0, hs1, h20, h21, W2, Wa, Wlr, blr,
                            targets.astype(jnp.float32), lp)
    return loss2[0, 0], pred


# speedup vs baseline: 9.6969x; 1.0339x over previous
import jax
import jax.numpy as jnp
import numpy as np
from jax import lax
from jax.experimental import pallas as pl
from jax.experimental.pallas import tpu as pltpu
from jax.experimental.pallas import tpu_sc as plsc

LNUM = 2
N = 50000
F = 128
H = 128
D = 128
B = 4096
S = 10

NC, NS = 2, 16
NW = NC * NS
BPW = B // NW
W1S = BPW * S
INV = 1.0 / (S + 1.0)
_CS = 32


_GBLK = 1000


def _g_body(feat_ref, w1_ref, g0_ref, g1_ref):
    dn = (((1,), (1,)), ((), ()))
    g0_ref[...] = lax.dot_general(
        feat_ref[0], w1_ref[0], dn, precision=lax.Precision.DEFAULT,
        preferred_element_type=jnp.float32) * INV
    g1_ref[...] = lax.dot_general(
        feat_ref[1], w1_ref[1], dn, precision=lax.Precision.DEFAULT,
        preferred_element_type=jnp.float32) * INV


def _pack_tables(neighbors):
    nbpad = jnp.concatenate(
        [neighbors, jnp.zeros((LNUM, N, 16 - S), jnp.int32)], axis=2)
    return nbpad.reshape(LNUM, N // 8, 128)


def _g_precompute(feat_tables, W1):
    return pl.pallas_call(
        _g_body,
        grid=(N // _GBLK,),
        in_specs=[
            pl.BlockSpec((LNUM, _GBLK, F), lambda i: (0, i, 0)),
            pl.BlockSpec((LNUM, H, F), lambda i: (0, 0, 0)),
        ],
        out_specs=[
            pl.BlockSpec((_GBLK, H), lambda i: (i, 0)),
            pl.BlockSpec((_GBLK, H), lambda i: (i, 0)),
        ],
        out_shape=[
            jax.ShapeDtypeStruct((N, H), jnp.float32),
            jax.ShapeDtypeStruct((N, H), jnp.float32),
        ],
    )(feat_tables, W1)


def _sum_chunk_item(selfg, nbg, i):
    out = []
    for v in range(H // 16):
        t = selfg[i, pl.ds(v * 16, 16)]
        for q in range(S):
            t = t + nbg[q * _CS + i, pl.ds(v * 16, 16)]
        out.append(jnp.maximum(t, 0.0))
    return out


def _zero_buf(buf):
    z = jnp.zeros((16,), jnp.float32)

    def zrow(i, c):
        for v in range(H // 16):
            buf[i, pl.ds(v * 16, 16)] = z
        return c

    lax.fori_loop(0, BPW, zrow, 0)


def _g_chunk_dmas(gt, self_idx_src, self_off, nb_idx_src, nb_base,
                  selfg, nbg, sem):
    soff = pl.multiple_of(self_off, _CS)
    ds = [pltpu.make_async_copy(
        gt.at[self_idx_src.at[pl.ds(soff, _CS)]], selfg, sem)]
    for s in range(S):
        off = pl.multiple_of(nb_base(s), _CS)
        ds.append(pltpu.make_async_copy(
            gt.at[nb_idx_src.at[pl.ds(off, _CS)]],
            nbg.at[pl.ds(s * _CS, _CS)], sem))
    return ds


def _extract_ids(src_ids, idrows, dst, dst_base, dst_stride):
    iota = lax.broadcasted_iota(jnp.int32, (16,), 0)

    def chunk(t, c):
        toff = pl.multiple_of(t * 16, 16)
        v = src_ids[pl.ds(toff, 16)]
        colb = (v & 7) * 16
        rows = iota + toff
        for s in range(S):
            ids = plsc.load_gather(idrows, [rows, colb + s])
            if ids.dtype != jnp.int32:
                ids = plsc.bitcast(ids, jnp.int32)
            off = pl.multiple_of(dst_base + s * dst_stride + toff, 16)
            dst[pl.ds(off, 16)] = ids
        return c

    lax.fori_loop(0, BPW // 16, chunk, 0)


def _fill_rowidx(src_ids, src_off, rowidx):
    def chunk(t, c):
        toff = pl.multiple_of(t * 16, 16)
        v = src_ids[pl.ds(src_off + toff, 16)]
        rowidx[pl.ds(toff, 16)] = lax.shift_right_logical(v, 3)
        return c

    lax.fori_loop(0, BPW // 16, chunk, 0)


def _pipelined_phase(gt, nch, self_src, self_off_fn, nb_src, nb_base_fn,
                     selfg2, nbg2, sem2, compute_fn):
    def dmas(c, p):
        return _g_chunk_dmas(
            gt, self_src, self_off_fn(c), nb_src,
            lambda s, c=c: nb_base_fn(c, s),
            selfg2.at[p], nbg2.at[p], sem2.at[p])

    for d in dmas(0, 0):
        d.start()

    def step(c, carry):
        p = lax.rem(c, 2)
        for d in dmas(c, p):
            d.wait()

        @pl.when(c + 1 < nch)
        def _():
            for d in dmas(c + 1, 1 - p):
                d.start()

        compute_fn(c, selfg2.at[p], nbg2.at[p])
        return carry

    lax.fori_loop(0, nch, step, 0)


def _sc_body(nodes_hbm, pkt, g0, g1, hs0, hs1, h20, h21,
             nodes_v, nb1w, nb2w, rowidx, selfg, nbg, hbuf, sem):
    wid = lax.axis_index("s") * NC + lax.axis_index("c")
    base = pl.multiple_of(wid * BPW, BPW)
    pltpu.sync_copy(nodes_hbm.at[pl.ds(base, BPW)], nodes_v)
    idrows_dma = [nbg.at[q, pl.ds(0, BPW)].bitcast(jnp.int32)
                  for q in range(2)]
    idrows = [nbg.at[q, pl.ds(0, BPW)] for q in range(2)]

    def id_gather(pk, q):
        return pltpu.make_async_copy(
            pk.at[rowidx.at[q]], idrows_dma[q], sem.at[q])

    for pk, gt, hs_out, h2_out in ((pkt.at[0], g0, hs0, h20),
                                   (pkt.at[1], g1, hs1, h21)):
        _fill_rowidx(nodes_v, 0, rowidx.at[0])
        id_gather(pk, 0).start()
        id_gather(pk, 0).wait()
        _extract_ids(nodes_v, idrows[0], nb1w, 0, BPW)

        _fill_rowidx(nb1w, 0, rowidx.at[0])
        id_gather(pk, 0).start()
        for k in range(W1S // BPW):
            q = k % 2
            if k + 1 < W1S // BPW:
                _fill_rowidx(nb1w, (k + 1) * BPW, rowidx.at[1 - q])
                id_gather(pk, 1 - q).start()
            id_gather(pk, q).wait()
            _extract_ids(nb1w.at[pl.ds(k * BPW, BPW)], idrows[q],
                         nb2w, k * BPW, W1S)

        def self_compute(c, sg, ng):
            def item(i, cc, c=c):
                vals = _sum_chunk_item(sg, ng, i)
                row = c * _CS + i
                for v in range(H // 16):
                    hbuf[row, pl.ds(v * 16, 16)] = vals[v]
                return cc

            lax.fori_loop(0, _CS, item, 0)

        _pipelined_phase(
            gt, BPW // _CS,
            nodes_v, lambda c: c * _CS,
            nb1w, lambda c, s: s * BPW + c * _CS,
            selfg, nbg, sem, self_compute)
        pltpu.sync_copy(hbuf, hs_out.at[pl.ds(base, BPW)])

        _zero_buf(hbuf)

        def nb_compute(c, sg, ng):
            def item(i, cc, c=c):
                vals = _sum_chunk_item(sg, ng, i)
                brow = lax.rem(c, BPW // _CS) * _CS + i
                for v in range(H // 16):
                    plsc.addupdate(hbuf.at[brow, pl.ds(v * 16, 16)], vals[v])
                return cc

            lax.fori_loop(0, _CS, item, 0)

        _pipelined_phase(
            gt, W1S // _CS,
            nb1w, lambda c: c * _CS,
            nb2w, lambda c, s: s * W1S + c * _CS,
            selfg, nbg, sem, nb_compute)
        pltpu.sync_copy(hbuf, h2_out.at[pl.ds(base, BPW)])


def _sc_aggregate(nodes, pkt, g0, g1):
    mesh = plsc.VectorSubcoreMesh(
        core_axis_name="c", subcore_axis_name="s",
        num_cores=NC, num_subcores=NS)
    out = jax.ShapeDtypeStruct((B, H), jnp.float32)
    f = pl.kernel(
        _sc_body,
        out_type=(out, out, out, out),
        mesh=mesh,
        scratch_types=[
            pltpu.VMEM((BPW,), jnp.int32),
            pltpu.VMEM((W1S,), jnp.int32),
            pltpu.VMEM((W1S * S,), jnp.int32),
            pltpu.VMEM((2, BPW), jnp.int32),
            pltpu.VMEM((2, _CS, H), jnp.float32),
            pltpu.VMEM((2, S * _CS, H), jnp.float32),
            pltpu.VMEM((BPW, H), jnp.float32),
            pltpu.SemaphoreType.DMA((2,)),
        ],
        compiler_params=pltpu.CompilerParams(needs_layout_passes=False),
    )
    return f(nodes, pkt, g0, g1)


def _fin_body(hs0, hs1, h20, h21, w2_ref, wa_ref, wlr_ref, blr_ref,
              tgt_ref, lp_ref, loss_ref, pred_ref):
    dn_t = (((1,), (1,)), ((), ()))
    dn = (((1,), (0,)), ((), ()))
    hp = lax.Precision.HIGHEST
    lp = lp_ref[0]

    embs = []
    for l, (hs, h2) in enumerate(((hs0, h20), (hs1, h21))):
        agg2 = (h2[...] + hs[...]) * INV
        emb = jnp.maximum(
            lax.dot_general(agg2, w2_ref[l], dn_t, precision=hp,
                            preferred_element_type=jnp.float32), 0.0)
        embs.append(emb)

    q = jnp.where(lp == 0, embs[0], embs[1])
    tq = jnp.tanh(q)
    scale = 1.0 / np.sqrt(D)
    scores = []
    for l in range(LNUM):
        proj = lax.dot_general(embs[l], wa_ref[...], dn, precision=hp,
                               preferred_element_type=jnp.float32)
        scores.append(
            jnp.sum(tq * jnp.tanh(proj), axis=1, keepdims=True) * scale)

    m = jnp.maximum(scores[0], scores[1])
    e0 = jnp.exp(scores[0] - m)
    e1 = jnp.exp(scores[1] - m)
    denom = e0 + e1
    fused = (e0 * embs[0] + e1 * embs[1]) / denom

    wlr = jnp.where(lp == 0, wlr_ref[0], wlr_ref[1])
    blr = jnp.where(lp == 0, blr_ref[0, 0], blr_ref[1, 0])
    logits = jnp.sum(fused * wlr, axis=1, keepdims=True) + blr
    pred = 1.0 / (1.0 + jnp.exp(-logits))
    pred_ref[...] = pred

    eps = 1e-7
    p = jnp.clip(pred, eps, 1.0 - eps)
    t = tgt_ref[...]
    ll = t * jnp.log(p) + (1.0 - t) * jnp.log(1.0 - p)
    loss_ref[...] = (-jnp.sum(ll) * (1.0 / B)).reshape(1, 1)


def _finalize(hs0, hs1, h20, h21, W2, Wa, Wlr, blr, targets, lp):
    return pl.pallas_call(
        _fin_body,
        in_specs=[pl.BlockSpec((B, H), lambda: (0, 0))] * 4 + [
            pl.BlockSpec((LNUM, D, H), lambda: (0, 0, 0)),
            pl.BlockSpec((D, D), lambda: (0, 0)),
            pl.BlockSpec((LNUM, 1, D), lambda: (0, 0, 0)),
            pl.BlockSpec((LNUM, 1), lambda: (0, 0)),
            pl.BlockSpec((B, 1), lambda: (0, 0)),
            pl.BlockSpec(memory_space=pltpu.SMEM),
        ],
        out_specs=[
            pl.BlockSpec((1, 1), lambda: (0, 0)),
            pl.BlockSpec((B, 1), lambda: (0, 0)),
        ],
        out_shape=[
            jax.ShapeDtypeStruct((1, 1), jnp.float32),
            jax.ShapeDtypeStruct((B, 1), jnp.float32),
        ],
    )(hs0, hs1, h20, h21, W2, Wa, Wlr, blr, targets, lp)


def kernel(nodes, targets, layer_predict, neighbors, feat_tables, W1, W2,
           Wa, Wlr, blr):
    nodes = nodes.astype(jnp.int32)
    neighbors = neighbors.astype(jnp.int32)
    pkt = _pack_tables(neighbors)
    g0, g1 = _g_precompute(feat_tables, W1)
    hs0, hs1, h20, h21 = _sc_aggregate(nodes, pkt, g0, g1)
    lp = jnp.asarray(layer_predict, jnp.int32).reshape(1)
    loss2, pred = _finalize(hs0, hs1, h20, h21, W2, Wa, Wlr, blr,
                            targets.astype(jnp.float32), lp)
    return loss2[0, 0], pred


# speedup vs baseline: 10.3095x; 1.0632x over previous
import jax
import jax.numpy as jnp
import numpy as np
from jax import lax
from jax.experimental import pallas as pl
from jax.experimental.pallas import tpu as pltpu
from jax.experimental.pallas import tpu_sc as plsc

LNUM = 2
N = 50000
F = 128
H = 128
D = 128
B = 4096
S = 10

NC, NS = 2, 16
NW = NC * NS
BPW = B // NW
W1S = BPW * S
INV = 1.0 / (S + 1.0)
_CS = 32


_GBLK = 1000


def _g_body(feat_ref, w1_ref, g0_ref, g1_ref):
    dn = (((1,), (1,)), ((), ()))
    g0_ref[...] = lax.dot_general(
        feat_ref[0], w1_ref[0], dn, precision=lax.Precision.DEFAULT,
        preferred_element_type=jnp.float32) * INV
    g1_ref[...] = lax.dot_general(
        feat_ref[1], w1_ref[1], dn, precision=lax.Precision.DEFAULT,
        preferred_element_type=jnp.float32) * INV


def _pack_tables(neighbors):
    return jnp.pad(neighbors.reshape(LNUM, N // 8, 8 * S),
                   ((0, 0), (0, 0), (0, 128 - 8 * S)))


def _g_precompute(feat_tables, W1):
    return pl.pallas_call(
        _g_body,
        grid=(N // _GBLK,),
        in_specs=[
            pl.BlockSpec((LNUM, _GBLK, F), lambda i: (0, i, 0)),
            pl.BlockSpec((LNUM, H, F), lambda i: (0, 0, 0)),
        ],
        out_specs=[
            pl.BlockSpec((_GBLK, H), lambda i: (i, 0)),
            pl.BlockSpec((_GBLK, H), lambda i: (i, 0)),
        ],
        out_shape=[
            jax.ShapeDtypeStruct((N, H), jnp.float32),
            jax.ShapeDtypeStruct((N, H), jnp.float32),
        ],
    )(feat_tables, W1)


def _sum_chunk_item(selfg, nbg, i):
    out = []
    for v in range(H // 16):
        t = selfg[i, pl.ds(v * 16, 16)]
        for q in range(S):
            t = t + nbg[q * _CS + i, pl.ds(v * 16, 16)]
        out.append(jnp.maximum(t, 0.0))
    return out


def _zero_buf(buf):
    z = jnp.zeros((16,), jnp.float32)

    def zrow(i, c):
        for v in range(H // 16):
            buf[i, pl.ds(v * 16, 16)] = z
        return c

    lax.fori_loop(0, BPW, zrow, 0)


def _g_chunk_dmas(gt, self_idx_src, self_off, nb_idx_src, nb_base,
                  selfg, nbg, sem):
    soff = pl.multiple_of(self_off, _CS)
    ds = [pltpu.make_async_copy(
        gt.at[self_idx_src.at[pl.ds(soff, _CS)]], selfg, sem)]
    for s in range(S):
        off = pl.multiple_of(nb_base(s), _CS)
        ds.append(pltpu.make_async_copy(
            gt.at[nb_idx_src.at[pl.ds(off, _CS)]],
            nbg.at[pl.ds(s * _CS, _CS)], sem))
    return ds


def _extract_ids(src_ids, idrows, dst, dst_base, dst_stride):
    iota = lax.broadcasted_iota(jnp.int32, (16,), 0)

    def chunk(t, c):
        toff = pl.multiple_of(t * 16, 16)
        v = src_ids[pl.ds(toff, 16)]
        colb = (v & 7) * S
        rows = iota + toff
        for s in range(S):
            ids = plsc.load_gather(idrows, [rows, colb + s])
            if ids.dtype != jnp.int32:
                ids = plsc.bitcast(ids, jnp.int32)
            off = pl.multiple_of(dst_base + s * dst_stride + toff, 16)
            dst[pl.ds(off, 16)] = ids
        return c

    lax.fori_loop(0, BPW // 16, chunk, 0)


def _fill_rowidx(src_ids, src_off, rowidx):
    def chunk(t, c):
        toff = pl.multiple_of(t * 16, 16)
        v = src_ids[pl.ds(src_off + toff, 16)]
        rowidx[pl.ds(toff, 16)] = lax.shift_right_logical(v, 3)
        return c

    lax.fori_loop(0, BPW // 16, chunk, 0)


def _pipelined_phase(gt, nch, self_src, self_off_fn, nb_src, nb_base_fn,
                     selfg2, nbg2, sem2, compute_fn):
    def dmas(c, p):
        return _g_chunk_dmas(
            gt, self_src, self_off_fn(c), nb_src,
            lambda s, c=c: nb_base_fn(c, s),
            selfg2.at[p], nbg2.at[p], sem2.at[p])

    for d in dmas(0, 0):
        d.start()

    def step(c, carry):
        p = lax.rem(c, 2)
        for d in dmas(c, p):
            d.wait()

        @pl.when(c + 1 < nch)
        def _():
            for d in dmas(c + 1, 1 - p):
                d.start()

        compute_fn(c, selfg2.at[p], nbg2.at[p])
        return carry

    lax.fori_loop(0, nch, step, 0)


def _sc_body(nodes_hbm, pkt, g0, g1, hs0, hs1, h20, h21,
             nodes_v, nb1w, nb2w, rowidx, selfg, nbg, hbuf, sem):
    wid = lax.axis_index("s") * NC + lax.axis_index("c")
    base = pl.multiple_of(wid * BPW, BPW)
    pltpu.sync_copy(nodes_hbm.at[pl.ds(base, BPW)], nodes_v)
    idrows_dma = [nbg.at[q, pl.ds(0, BPW)].bitcast(jnp.int32)
                  for q in range(2)]
    idrows = [nbg.at[q, pl.ds(0, BPW)] for q in range(2)]

    def id_gather(pk, q):
        return pltpu.make_async_copy(
            pk.at[rowidx.at[q]], idrows_dma[q], sem.at[q])

    for pk, gt, hs_out, h2_out in ((pkt.at[0], g0, hs0, h20),
                                   (pkt.at[1], g1, hs1, h21)):
        _fill_rowidx(nodes_v, 0, rowidx.at[0])
        id_gather(pk, 0).start()
        id_gather(pk, 0).wait()
        _extract_ids(nodes_v, idrows[0], nb1w, 0, BPW)

        _fill_rowidx(nb1w, 0, rowidx.at[0])
        id_gather(pk, 0).start()
        for k in range(W1S // BPW):
            q = k % 2
            if k + 1 < W1S // BPW:
                _fill_rowidx(nb1w, (k + 1) * BPW, rowidx.at[1 - q])
                id_gather(pk, 1 - q).start()
            id_gather(pk, q).wait()
            _extract_ids(nb1w.at[pl.ds(k * BPW, BPW)], idrows[q],
                         nb2w, k * BPW, W1S)

        def self_compute(c, sg, ng):
            def item(i, cc, c=c):
                vals = _sum_chunk_item(sg, ng, i)
                row = c * _CS + i
                for v in range(H // 16):
                    hbuf[row, pl.ds(v * 16, 16)] = vals[v]
                return cc

            lax.fori_loop(0, _CS, item, 0)

        _pipelined_phase(
            gt, BPW // _CS,
            nodes_v, lambda c: c * _CS,
            nb1w, lambda c, s: s * BPW + c * _CS,
            selfg, nbg, sem, self_compute)
        pltpu.sync_copy(hbuf, hs_out.at[pl.ds(base, BPW)])

        _zero_buf(hbuf)

        def nb_compute(c, sg, ng):
            def item(i, cc, c=c):
                vals = _sum_chunk_item(sg, ng, i)
                brow = lax.rem(c, BPW // _CS) * _CS + i
                for v in range(H // 16):
                    plsc.addupdate(hbuf.at[brow, pl.ds(v * 16, 16)], vals[v])
                return cc

            lax.fori_loop(0, _CS, item, 0)

        _pipelined_phase(
            gt, W1S // _CS,
            nb1w, lambda c: c * _CS,
            nb2w, lambda c, s: s * W1S + c * _CS,
            selfg, nbg, sem, nb_compute)
        pltpu.sync_copy(hbuf, h2_out.at[pl.ds(base, BPW)])


def _sc_aggregate(nodes, pkt, g0, g1):
    mesh = plsc.VectorSubcoreMesh(
        core_axis_name="c", subcore_axis_name="s",
        num_cores=NC, num_subcores=NS)
    out = jax.ShapeDtypeStruct((B, H), jnp.float32)
    f = pl.kernel(
        _sc_body,
        out_type=(out, out, out, out),
        mesh=mesh,
        scratch_types=[
            pltpu.VMEM((BPW,), jnp.int32),
            pltpu.VMEM((W1S,), jnp.int32),
            pltpu.VMEM((W1S * S,), jnp.int32),
            pltpu.VMEM((2, BPW), jnp.int32),
            pltpu.VMEM((2, _CS, H), jnp.float32),
            pltpu.VMEM((2, S * _CS, H), jnp.float32),
            pltpu.VMEM((BPW, H), jnp.float32),
            pltpu.SemaphoreType.DMA((2,)),
        ],
        compiler_params=pltpu.CompilerParams(needs_layout_passes=False),
    )
    return f(nodes, pkt, g0, g1)


def _fin_body(hs0, hs1, h20, h21, w2_ref, wa_ref, wlr_ref, blr_ref,
              tgt_ref, lp_ref, loss_ref, pred_ref):
    dn_t = (((1,), (1,)), ((), ()))
    dn = (((1,), (0,)), ((), ()))
    hp = lax.Precision.HIGHEST
    lp = lp_ref[0]

    embs = []
    for l, (hs, h2) in enumerate(((hs0, h20), (hs1, h21))):
        agg2 = (h2[...] + hs[...]) * INV
        emb = jnp.maximum(
            lax.dot_general(agg2, w2_ref[l], dn_t, precision=hp,
                            preferred_element_type=jnp.float32), 0.0)
        embs.append(emb)

    q = jnp.where(lp == 0, embs[0], embs[1])
    tq = jnp.tanh(q)
    scale = 1.0 / np.sqrt(D)
    scores = []
    for l in range(LNUM):
        proj = lax.dot_general(embs[l], wa_ref[...], dn, precision=hp,
                               preferred_element_type=jnp.float32)
        scores.append(
            jnp.sum(tq * jnp.tanh(proj), axis=1, keepdims=True) * scale)

    m = jnp.maximum(scores[0], scores[1])
    e0 = jnp.exp(scores[0] - m)
    e1 = jnp.exp(scores[1] - m)
    denom = e0 + e1
    fused = (e0 * embs[0] + e1 * embs[1]) / denom

    wlr = jnp.where(lp == 0, wlr_ref[0], wlr_ref[1])
    blr = jnp.where(lp == 0, blr_ref[0, 0], blr_ref[1, 0])
    logits = jnp.sum(fused * wlr, axis=1, keepdims=True) + blr
    pred = 1.0 / (1.0 + jnp.exp(-logits))
    pred_ref[...] = pred

    eps = 1e-7
    p = jnp.clip(pred, eps, 1.0 - eps)
    t = tgt_ref[...]
    ll = t * jnp.log(p) + (1.0 - t) * jnp.log(1.0 - p)
    loss_ref[...] = (-jnp.sum(ll) * (1.0 / B)).reshape(1, 1)


def _finalize(hs0, hs1, h20, h21, W2, Wa, Wlr, blr, targets, lp):
    return pl.pallas_call(
        _fin_body,
        in_specs=[pl.BlockSpec((B, H), lambda: (0, 0))] * 4 + [
            pl.BlockSpec((LNUM, D, H), lambda: (0, 0, 0)),
            pl.BlockSpec((D, D), lambda: (0, 0)),
            pl.BlockSpec((LNUM, 1, D), lambda: (0, 0, 0)),
            pl.BlockSpec((LNUM, 1), lambda: (0, 0)),
            pl.BlockSpec((B, 1), lambda: (0, 0)),
            pl.BlockSpec(memory_space=pltpu.SMEM),
        ],
        out_specs=[
            pl.BlockSpec((1, 1), lambda: (0, 0)),
            pl.BlockSpec((B, 1), lambda: (0, 0)),
        ],
        out_shape=[
            jax.ShapeDtypeStruct((1, 1), jnp.float32),
            jax.ShapeDtypeStruct((B, 1), jnp.float32),
        ],
    )(hs0, hs1, h20, h21, W2, Wa, Wlr, blr, targets, lp)


def kernel(nodes, targets, layer_predict, neighbors, feat_tables, W1, W2,
           Wa, Wlr, blr):
    nodes = nodes.astype(jnp.int32)
    neighbors = neighbors.astype(jnp.int32)
    pkt = _pack_tables(neighbors)
    g0, g1 = _g_precompute(feat_tables, W1)
    hs0, hs1, h20, h21 = _sc_aggregate(nodes, pkt, g0, g1)
    lp = jnp.asarray(layer_predict, jnp.int32).reshape(1)
    loss2, pred = _finalize(hs0, hs1, h20, h21, W2, Wa, Wlr, blr,
                            targets.astype(jnp.float32), lp)
    return loss2[0, 0], pred


# speedup vs baseline: 10.7569x; 1.0434x over previous
import jax
import jax.numpy as jnp
import numpy as np
from jax import lax
from jax.experimental import pallas as pl
from jax.experimental.pallas import tpu as pltpu
from jax.experimental.pallas import tpu_sc as plsc

LNUM = 2
N = 50000
F = 128
H = 128
D = 128
B = 4096
S = 10

NC, NS = 2, 16
NW = NC * NS
BPW = B // NW
W1S = BPW * S
INV = 1.0 / (S + 1.0)
_CS = 32


_GBLK = 1000


def _g_body(feat_ref, w1_ref, g0_ref, g1_ref):
    dn = (((1,), (1,)), ((), ()))
    g0_ref[...] = lax.dot_general(
        feat_ref[0], w1_ref[0], dn, precision=lax.Precision.DEFAULT,
        preferred_element_type=jnp.float32) * INV
    g1_ref[...] = lax.dot_general(
        feat_ref[1], w1_ref[1], dn, precision=lax.Precision.DEFAULT,
        preferred_element_type=jnp.float32) * INV


def _pack_tables(neighbors):
    return jnp.pad(neighbors.reshape(LNUM, N // 8, 8 * S),
                   ((0, 0), (0, 0), (0, 128 - 8 * S)))


def _g_precompute(feat_tables, W1):
    return pl.pallas_call(
        _g_body,
        grid=(N // _GBLK,),
        in_specs=[
            pl.BlockSpec((LNUM, _GBLK, F), lambda i: (0, i, 0)),
            pl.BlockSpec((LNUM, H, F), lambda i: (0, 0, 0)),
        ],
        out_specs=[
            pl.BlockSpec((_GBLK, H), lambda i: (i, 0)),
            pl.BlockSpec((_GBLK, H), lambda i: (i, 0)),
        ],
        out_shape=[
            jax.ShapeDtypeStruct((N, H), jnp.float32),
            jax.ShapeDtypeStruct((N, H), jnp.float32),
        ],
    )(feat_tables, W1)


def _sum_chunk_item(selfg, nbg, i):
    out = []
    for v in range(H // 16):
        t = selfg[i, pl.ds(v * 16, 16)]
        for q in range(S):
            t = t + nbg[q * _CS + i, pl.ds(v * 16, 16)]
        out.append(jnp.maximum(t, 0.0))
    return out


def _zero_buf(buf):
    z = jnp.zeros((16,), jnp.float32)

    def zrow(i, c):
        for v in range(H // 16):
            buf[i, pl.ds(v * 16, 16)] = z
        return c

    lax.fori_loop(0, BPW, zrow, 0)


def _g_chunk_dmas(gt, self_idx_src, self_off, nb_idx_src, nb_base,
                  selfg, nbg, sem):
    soff = pl.multiple_of(self_off, _CS)
    ds = [pltpu.make_async_copy(
        gt.at[self_idx_src.at[pl.ds(soff, _CS)]], selfg, sem)]
    for s in range(S):
        off = pl.multiple_of(nb_base(s), _CS)
        ds.append(pltpu.make_async_copy(
            gt.at[nb_idx_src.at[pl.ds(off, _CS)]],
            nbg.at[pl.ds(s * _CS, _CS)], sem))
    return ds


def _extract_ids(src_ids, idrows, dst, dst_base, dst_stride):
    iota = lax.broadcasted_iota(jnp.int32, (16,), 0)

    def chunk(t, c):
        toff = pl.multiple_of(t * 16, 16)
        v = src_ids[pl.ds(toff, 16)]
        colb = (v & 7) * S
        rows = iota + toff
        for s in range(S):
            ids = plsc.load_gather(idrows, [rows, colb + s])
            if ids.dtype != jnp.int32:
                ids = plsc.bitcast(ids, jnp.int32)
            off = pl.multiple_of(dst_base + s * dst_stride + toff, 16)
            dst[pl.ds(off, 16)] = ids
        return c

    lax.fori_loop(0, BPW // 16, chunk, 0)


def _fill_rowidx(src_ids, src_off, rowidx):
    def chunk(t, c):
        toff = pl.multiple_of(t * 16, 16)
        v = src_ids[pl.ds(src_off + toff, 16)]
        rowidx[pl.ds(toff, 16)] = lax.shift_right_logical(v, 3)
        return c

    lax.fori_loop(0, BPW // 16, chunk, 0)


def _pipelined_phase(gt, nch, self_src, self_off_fn, nb_src, nb_base_fn,
                     selfg2, nbg2, sem2, compute_fn):
    def dmas(c, p):
        return _g_chunk_dmas(
            gt, self_src, self_off_fn(c), nb_src,
            lambda s, c=c: nb_base_fn(c, s),
            selfg2.at[p], nbg2.at[p], sem2.at[p])

    for d in dmas(0, 0):
        d.start()

    def step(c, carry):
        p = lax.rem(c, 2)
        for d in dmas(c, p):
            d.wait()

        @pl.when(c + 1 < nch)
        def _():
            for d in dmas(c + 1, 1 - p):
                d.start()

        compute_fn(c, selfg2.at[p], nbg2.at[p])
        return carry

    lax.fori_loop(0, nch, step, 0)


def _sc_id_body(nodes_hbm, pkt, nb1o0, nb1o1, nb2o0, nb2o1,
                nodes_v, nb1w, nb2w, rowidx, idrows, sem):
    wid = lax.axis_index("s") * NC + lax.axis_index("c")
    base = pl.multiple_of(wid * BPW, BPW)
    pltpu.sync_copy(nodes_hbm.at[pl.ds(base, BPW)], nodes_v)

    def id_gather(pk, q):
        return pltpu.make_async_copy(
            pk.at[rowidx.at[q]], idrows.at[q], sem.at[q])

    for pk, nb1_out, nb2_out in ((pkt.at[0], nb1o0, nb2o0),
                                 (pkt.at[1], nb1o1, nb2o1)):
        _fill_rowidx(nodes_v, 0, rowidx.at[0])
        id_gather(pk, 0).start()
        id_gather(pk, 0).wait()
        _extract_ids(nodes_v, idrows.at[0], nb1w, 0, BPW)

        _fill_rowidx(nb1w, 0, rowidx.at[0])
        id_gather(pk, 0).start()
        for k in range(W1S // BPW):
            q = k % 2
            if k + 1 < W1S // BPW:
                _fill_rowidx(nb1w, (k + 1) * BPW, rowidx.at[1 - q])
                id_gather(pk, 1 - q).start()
            id_gather(pk, q).wait()
            _extract_ids(nb1w.at[pl.ds(k * BPW, BPW)], idrows.at[q],
                         nb2w, k * BPW, W1S)

        pltpu.sync_copy(nb1w, nb1_out.at[pl.ds(wid * W1S, W1S)])
        pltpu.sync_copy(nb2w, nb2_out.at[pl.ds(wid * W1S * S, W1S * S)])


def _sc_id(nodes, pkt):
    mesh = plsc.VectorSubcoreMesh(
        core_axis_name="c", subcore_axis_name="s",
        num_cores=NC, num_subcores=NS)
    o1 = jax.ShapeDtypeStruct((B * S,), jnp.int32)
    o2 = jax.ShapeDtypeStruct((B * S * S,), jnp.int32)
    f = pl.kernel(
        _sc_id_body,
        out_type=(o1, o1, o2, o2),
        mesh=mesh,
        scratch_types=[
            pltpu.VMEM((BPW,), jnp.int32),
            pltpu.VMEM((W1S,), jnp.int32),
            pltpu.VMEM((W1S * S,), jnp.int32),
            pltpu.VMEM((2, BPW), jnp.int32),
            pltpu.VMEM((2, BPW, 128), jnp.int32),
            pltpu.SemaphoreType.DMA((2,)),
        ],
        compiler_params=pltpu.CompilerParams(needs_layout_passes=False),
    )
    return f(nodes, pkt)


def _sc_body(nodes_hbm, nb1o0, nb1o1, nb2o0, nb2o1, g0, g1,
             hs0, hs1, h20, h21,
             nodes_v, nb1w, nb2w, selfg, nbg, hbuf, sem):
    wid = lax.axis_index("s") * NC + lax.axis_index("c")
    base = pl.multiple_of(wid * BPW, BPW)
    pltpu.sync_copy(nodes_hbm.at[pl.ds(base, BPW)], nodes_v)

    for nb1_in, nb2_in, gt, hs_out, h2_out in (
            (nb1o0, nb2o0, g0, hs0, h20),
            (nb1o1, nb2o1, g1, hs1, h21)):
        pltpu.sync_copy(nb1_in.at[pl.ds(wid * W1S, W1S)], nb1w)
        pltpu.sync_copy(nb2_in.at[pl.ds(wid * W1S * S, W1S * S)], nb2w)

        def self_compute(c, sg, ng):
            def item(i, cc, c=c):
                vals = _sum_chunk_item(sg, ng, i)
                row = c * _CS + i
                for v in range(H // 16):
                    hbuf[row, pl.ds(v * 16, 16)] = vals[v]
                return cc

            lax.fori_loop(0, _CS, item, 0)

        _pipelined_phase(
            gt, BPW // _CS,
            nodes_v, lambda c: c * _CS,
            nb1w, lambda c, s: s * BPW + c * _CS,
            selfg, nbg, sem, self_compute)
        pltpu.sync_copy(hbuf, hs_out.at[pl.ds(base, BPW)])

        _zero_buf(hbuf)

        def nb_compute(c, sg, ng):
            def item(i, cc, c=c):
                vals = _sum_chunk_item(sg, ng, i)
                brow = lax.rem(c, BPW // _CS) * _CS + i
                for v in range(H // 16):
                    plsc.addupdate(hbuf.at[brow, pl.ds(v * 16, 16)], vals[v])
                return cc

            lax.fori_loop(0, _CS, item, 0)

        _pipelined_phase(
            gt, W1S // _CS,
            nb1w, lambda c: c * _CS,
            nb2w, lambda c, s: s * W1S + c * _CS,
            selfg, nbg, sem, nb_compute)
        pltpu.sync_copy(hbuf, h2_out.at[pl.ds(base, BPW)])


def _sc_aggregate(nodes, nb1o0, nb1o1, nb2o0, nb2o1, g0, g1):
    mesh = plsc.VectorSubcoreMesh(
        core_axis_name="c", subcore_axis_name="s",
        num_cores=NC, num_subcores=NS)
    out = jax.ShapeDtypeStruct((B, H), jnp.float32)
    f = pl.kernel(
        _sc_body,
        out_type=(out, out, out, out),
        mesh=mesh,
        scratch_types=[
            pltpu.VMEM((BPW,), jnp.int32),
            pltpu.VMEM((W1S,), jnp.int32),
            pltpu.VMEM((W1S * S,), jnp.int32),
            pltpu.VMEM((2, _CS, H), jnp.float32),
            pltpu.VMEM((2, S * _CS, H), jnp.float32),
            pltpu.VMEM((BPW, H), jnp.float32),
            pltpu.SemaphoreType.DMA((2,)),
        ],
        compiler_params=pltpu.CompilerParams(needs_layout_passes=False),
    )
    return f(nodes, nb1o0, nb1o1, nb2o0, nb2o1, g0, g1)


def _fin_body(hs0, hs1, h20, h21, w2_ref, wa_ref, wlr_ref, blr_ref,
              tgt_ref, lp_ref, loss_ref, pred_ref):
    dn_t = (((1,), (1,)), ((), ()))
    dn = (((1,), (0,)), ((), ()))
    hp = lax.Precision.HIGHEST
    lp = lp_ref[0]

    embs = []
    for l, (hs, h2) in enumerate(((hs0, h20), (hs1, h21))):
        agg2 = (h2[...] + hs[...]) * INV
        emb = jnp.maximum(
            lax.dot_general(agg2, w2_ref[l], dn_t, precision=hp,
                            preferred_element_type=jnp.float32), 0.0)
        embs.append(emb)

    q = jnp.where(lp == 0, embs[0], embs[1])
    tq = jnp.tanh(q)
    scale = 1.0 / np.sqrt(D)
    scores = []
    for l in range(LNUM):
        proj = lax.dot_general(embs[l], wa_ref[...], dn, precision=hp,
                               preferred_element_type=jnp.float32)
        scores.append(
            jnp.sum(tq * jnp.tanh(proj), axis=1, keepdims=True) * scale)

    m = jnp.maximum(scores[0], scores[1])
    e0 = jnp.exp(scores[0] - m)
    e1 = jnp.exp(scores[1] - m)
    denom = e0 + e1
    fused = (e0 * embs[0] + e1 * embs[1]) / denom

    wlr = jnp.where(lp == 0, wlr_ref[0], wlr_ref[1])
    blr = jnp.where(lp == 0, blr_ref[0, 0], blr_ref[1, 0])
    logits = jnp.sum(fused * wlr, axis=1, keepdims=True) + blr
    pred = 1.0 / (1.0 + jnp.exp(-logits))
    pred_ref[...] = pred

    eps = 1e-7
    p = jnp.clip(pred, eps, 1.0 - eps)
    t = tgt_ref[...]
    ll = t * jnp.log(p) + (1.0 - t) * jnp.log(1.0 - p)
    loss_ref[...] = (-jnp.sum(ll) * (1.0 / B)).reshape(1, 1)


def _finalize(hs0, hs1, h20, h21, W2, Wa, Wlr, blr, targets, lp):
    return pl.pallas_call(
        _fin_body,
        in_specs=[pl.BlockSpec((B, H), lambda: (0, 0))] * 4 + [
            pl.BlockSpec((LNUM, D, H), lambda: (0, 0, 0)),
            pl.BlockSpec((D, D), lambda: (0, 0)),
            pl.BlockSpec((LNUM, 1, D), lambda: (0, 0, 0)),
            pl.BlockSpec((LNUM, 1), lambda: (0, 0)),
            pl.BlockSpec((B, 1), lambda: (0, 0)),
            pl.BlockSpec(memory_space=pltpu.SMEM),
        ],
        out_specs=[
            pl.BlockSpec((1, 1), lambda: (0, 0)),
            pl.BlockSpec((B, 1), lambda: (0, 0)),
        ],
        out_shape=[
            jax.ShapeDtypeStruct((1, 1), jnp.float32),
            jax.ShapeDtypeStruct((B, 1), jnp.float32),
        ],
    )(hs0, hs1, h20, h21, W2, Wa, Wlr, blr, targets, lp)


def kernel(nodes, targets, layer_predict, neighbors, feat_tables, W1, W2,
           Wa, Wlr, blr):
    nodes = nodes.astype(jnp.int32)
    neighbors = neighbors.astype(jnp.int32)
    pkt = _pack_tables(neighbors)
    nb1o0, nb1o1, nb2o0, nb2o1 = _sc_id(nodes, pkt)
    g0, g1 = _g_precompute(feat_tables, W1)
    hs0, hs1, h20, h21 = _sc_aggregate(nodes, nb1o0, nb1o1, nb2o0, nb2o1,
                                       g0, g1)
    lp = jnp.asarray(layer_predict, jnp.int32).reshape(1)
    loss2, pred = _finalize(hs0, hs1, h20, h21, W2, Wa, Wlr, blr,
                            targets.astype(jnp.float32), lp)
    return loss2[0, 0], pred
